# baseline (speedup 1.0000x reference)
import functools

import jax
import jax.numpy as jnp
from jax.experimental import pallas as pl
from jax.experimental.pallas import tpu as pltpu

_B, _S, _D, _E = 2, 2048, 1024, 8
_TS = 512


def _router_kernel(h_ref, qw_ref, cw_ref, out_ref, *, n_s_tiles):
    s = pl.program_id(1)

    x = h_ref[0]
    q = jax.lax.dot_general(
        x, qw_ref[...], (((1,), (1,)), ((), ())),
        precision=jax.lax.Precision.DEFAULT,
        preferred_element_type=jnp.float32)
    qn2 = jnp.sum(q * q, axis=1, keepdims=True)

    lane = jax.lax.broadcasted_iota(jnp.int32, (1, _E), 1)
    acc = jnp.zeros((1, _E), dtype=jnp.float32)
    for e in range(_E):
        k = jax.lax.dot_general(
            x, cw_ref[e], (((1,), (1,)), ((), ())),
            precision=jax.lax.Precision.DEFAULT,
            preferred_element_type=jnp.float32)
        part = jnp.sum(jnp.sum(k, axis=0, keepdims=True)) * (1.0 / _S)
        acc = acc + jnp.where(lane == e, part, 0.0)

    @pl.when(s == 0)
    def _init():
        out_ref[...] = jnp.zeros_like(out_ref)

    out_ref[...] += acc[None]


@jax.jit
def kernel(h, q_W, chip_weights):
    n_s_tiles = _S // _TS
    out = pl.pallas_call(
        functools.partial(_router_kernel, n_s_tiles=n_s_tiles),
        grid=(_B, n_s_tiles),
        in_specs=[
            pl.BlockSpec((1, _TS, _D), lambda b, s: (b, s, 0)),
            pl.BlockSpec((_D, _D), lambda b, s: (0, 0)),
            pl.BlockSpec((_E, _D, _D), lambda b, s: (0, 0, 0)),
        ],
        out_specs=pl.BlockSpec((1, 1, _E), lambda b, s: (b, 0, 0)),
        out_shape=jax.ShapeDtypeStruct((_B, 1, _E), jnp.float32),
        compiler_params=pltpu.CompilerParams(
            dimension_semantics=("parallel", "arbitrary"),
        ),
    )(h, q_W, chip_weights)
    return out.reshape(_B, _E)


# speedup vs baseline: 2.0182x; 2.0182x over previous
import functools

import jax
import jax.numpy as jnp
from jax.experimental import pallas as pl
from jax.experimental.pallas import tpu as pltpu

_B, _S, _D, _E = 2, 2048, 1024, 8
_TS = 1024


def _router_kernel(h_ref, qw_ref, cw_ref, out_ref, *, n_s_tiles):
    s = pl.program_id(1)

    x = h_ref[0]
    q = jax.lax.dot_general(
        x, qw_ref[...], (((1,), (1,)), ((), ())),
        precision=jax.lax.Precision.DEFAULT,
        preferred_element_type=jnp.float32)
    qn2 = jnp.sum(q * q, axis=1, keepdims=True)

    lane = jax.lax.broadcasted_iota(jnp.int32, (1, _E), 1)
    acc = jnp.zeros((1, _E), dtype=jnp.float32)
    for e in range(_E):
        k = jax.lax.dot_general(
            x, cw_ref[e], (((1,), (1,)), ((), ())),
            precision=jax.lax.Precision.DEFAULT,
            preferred_element_type=jnp.float32)
        num = jnp.sum(q * k, axis=1, keepdims=True)
        kn2 = jnp.sum(k * k, axis=1, keepdims=True)
        d2 = qn2 * kn2
        inv = jnp.where(d2 <= 1e-16, 1e8, jax.lax.rsqrt(d2))
        part = jnp.sum(num * inv) * (1.0 / _S)
        acc = acc + jnp.where(lane == e, part, 0.0)

    @pl.when(s == 0)
    def _init():
        out_ref[...] = jnp.zeros_like(out_ref)

    out_ref[...] += acc[None]


@jax.jit
def kernel(h, q_W, chip_weights):
    n_s_tiles = _S // _TS
    out = pl.pallas_call(
        functools.partial(_router_kernel, n_s_tiles=n_s_tiles),
        grid=(_B, n_s_tiles),
        in_specs=[
            pl.BlockSpec((1, _TS, _D), lambda b, s: (b, s, 0)),
            pl.BlockSpec((_D, _D), lambda b, s: (0, 0)),
            pl.BlockSpec((_E, _D, _D), lambda b, s: (0, 0, 0)),
        ],
        out_specs=pl.BlockSpec((1, 1, _E), lambda b, s: (b, 0, 0)),
        out_shape=jax.ShapeDtypeStruct((_B, 1, _E), jnp.float32),
        compiler_params=pltpu.CompilerParams(
            dimension_semantics=("parallel", "arbitrary"),
        ),
    )(h, q_W, chip_weights)
    return out.reshape(_B, _E)
